# Initial kernel scaffold; baseline (speedup 1.0000x reference)
import jax
import jax.numpy as jnp
from jax.experimental import pallas as pl


def kernel(user_emb, item_emb, user_idx, item_idx):
    raise NotImplementedError("write your pallas kernel here")


# speedup vs baseline: 4.1050x; 4.1050x over previous
import functools

import jax
import jax.numpy as jnp
from jax import lax
from jax.experimental import pallas as pl
from jax.experimental.pallas import tpu as pltpu
from jax.experimental.pallas import tpu_sc as plsc

_U = 50001
_I = 50001
_N = _U + _I
_E2 = 1600000
_D = 64

_BLK = 128
_NSUB = 16
_NCORE = 2

_EPT_SEG = 100096
_NBLK_SEG = _EPT_SEG // _BLK
_EPAD = _EPT_SEG * _NSUB
_EPT_DEG = _EPAD // 32
_NBLK_DEG = _EPT_DEG // _BLK

_CHUNK = 28672
_NCHUNK = 4
_NPAD = _CHUNK * _NCHUNK
_STRIPE = _CHUNK // _NSUB
_DEG_W = 8
_DEG_STRIPE = _NPAD // _NSUB


def _mesh():
    return plsc.VectorSubcoreMesh(core_axis_name="c", subcore_axis_name="s")


@functools.partial(
    pl.kernel,
    mesh=_mesh(),
    compiler_params=pltpu.CompilerParams(use_tc_tiling_on_sc=False),
    out_type=jax.ShapeDtypeStruct((_NCORE * _NPAD, _DEG_W), jnp.float32),
    scratch_types=[
        pltpu.VMEM((_BLK,), jnp.int32),
        pltpu.VMEM((_BLK, _DEG_W), jnp.float32),
        pltpu.VMEM_SHARED((_NPAD, _DEG_W), jnp.float32),
    ],
)
def _deg_kernel(rows_hbm, ones_hbm, zeros_hbm, out_hbm, ridx_v, ones_v, acc):
    c = lax.axis_index("c")
    s = lax.axis_index("s")
    pltpu.sync_copy(ones_hbm, ones_v)
    pltpu.sync_copy(zeros_hbm, acc.at[pl.ds(s * _DEG_STRIPE, _DEG_STRIPE)])
    plsc.subcore_barrier()
    base = (s * _NCORE + c) * _EPT_DEG

    def body(b, carry):
        pltpu.sync_copy(rows_hbm.at[pl.ds(base + b * _BLK, _BLK)], ridx_v)
        pltpu.sync_copy(ones_v, acc.at[ridx_v], add=True)
        return carry

    lax.fori_loop(0, _NBLK_DEG, body, 0)
    plsc.subcore_barrier()
    pltpu.sync_copy(
        acc.at[pl.ds(s * _DEG_STRIPE, _DEG_STRIPE)],
        out_hbm.at[pl.ds(c * _NPAD + s * _DEG_STRIPE, _DEG_STRIPE)],
    )


@functools.partial(
    pl.kernel,
    mesh=_mesh(),
    compiler_params=pltpu.CompilerParams(use_tc_tiling_on_sc=False),
    out_type=jax.ShapeDtypeStruct((_NPAD, _D), jnp.float32),
    scratch_types=[
        pltpu.VMEM((_BLK,), jnp.int32),
        pltpu.VMEM((_BLK,), jnp.int32),
        pltpu.VMEM((_BLK,), jnp.int32),
        pltpu.VMEM((_BLK, _D), jnp.float32),
        pltpu.SemaphoreType.DMA,
        pltpu.VMEM_SHARED((_CHUNK + 8, _D), jnp.float32),
    ],
)
def _seg_kernel(x_hbm, rows_hbm, cols_hbm, zeros_hbm, out_hbm,
                ridx_v, cidx_v, lidx_v, g_v, sem, acc):
    c = lax.axis_index("c")
    s = lax.axis_index("s")
    base = s * _EPT_SEG
    for i in range(_NCHUNK // _NCORE):
        ko = c + _NCORE * i
        lo = ko * _CHUNK
        pltpu.sync_copy(zeros_hbm, acc.at[pl.ds(s * _STRIPE, _STRIPE)])
        plsc.subcore_barrier()
        lo_v = jnp.full((16,), lo, jnp.int32)
        hi_v = lo_v + _CHUNK

        def body(b, carry):
            off = base + b * _BLK
            pltpu.sync_copy(rows_hbm.at[pl.ds(off, _BLK)], ridx_v)
            pltpu.sync_copy(cols_hbm.at[pl.ds(off, _BLK)], cidx_v)
            for j in range(_BLK // 16):
                r = ridx_v[pl.ds(j * 16, 16)]
                ok = (r >= lo_v) & (r < hi_v)
                lidx_v[pl.ds(j * 16, 16)] = jnp.where(ok, r - lo_v, _CHUNK)
            pltpu.async_copy(x_hbm.at[cidx_v], g_v, sem).wait()
            pltpu.sync_copy(g_v, acc.at[lidx_v], add=True)
            return carry

        lax.fori_loop(0, _NBLK_SEG, body, 0)
        plsc.subcore_barrier()
        pltpu.sync_copy(
            acc.at[pl.ds(s * _STRIPE, _STRIPE)],
            out_hbm.at[pl.ds(lo + s * _STRIPE, _STRIPE)],
        )
        plsc.subcore_barrier()


def kernel(user_emb, item_emb, user_idx, item_idx):
    rows = jnp.concatenate([user_idx, item_idx + _U])
    cols = jnp.concatenate([item_idx + _U, user_idx])
    pad = _EPAD - _E2
    rows_p = jnp.concatenate([rows, jnp.full((pad,), _N, jnp.int32)])
    cols_p = jnp.concatenate([cols, jnp.zeros((pad,), jnp.int32)])

    ones_blk = jnp.ones((_BLK, _DEG_W), jnp.float32)
    zeros_deg = jnp.zeros((_DEG_STRIPE, _DEG_W), jnp.float32)
    deg_parts = _deg_kernel(rows_p, ones_blk, zeros_deg)
    deg = deg_parts[: _NPAD, 0] + deg_parts[_NPAD:, 0]
    dis = jax.lax.rsqrt(deg[:_N] + 1e-07)

    e0 = jnp.concatenate([user_emb, item_emb], axis=0)
    zeros_seg = jnp.zeros((_STRIPE, _D), jnp.float32)
    out_acc = e0
    x = e0
    for _ in range(3):
        xs = dis[:, None] * x
        y = _seg_kernel(xs, rows_p, cols_p, zeros_seg)
        x = dis[:, None] * y[:_N]
        out_acc = out_acc + x
    total = out_acc * 0.25
    return (total[:_U], total[_U:])
